# baseline (speedup 1.0000x reference)
import jax
import jax.numpy as jnp
from jax.experimental import pallas as pl
from jax.experimental.pallas import tpu as pltpu

_CHUNKS = 4
_ROWS = 32 // _CHUNKS


def _copy_body(k_hbm, v_hbm, ok_hbm, ov_hbm, kb, vb, sem_in, sem_out):
    def sl(i):
        return pl.ds(i * _ROWS, _ROWS)

    ins = []
    for i in range(_CHUNKS):
        ins.append(pltpu.make_async_copy(k_hbm.at[sl(i)], kb.at[sl(i)], sem_in.at[2 * i]))
        ins.append(pltpu.make_async_copy(v_hbm.at[sl(i)], vb.at[sl(i)], sem_in.at[2 * i + 1]))
    outs = []
    for i in range(_CHUNKS):
        outs.append(pltpu.make_async_copy(kb.at[sl(i)], ok_hbm.at[sl(i)], sem_out.at[2 * i]))
        outs.append(pltpu.make_async_copy(vb.at[sl(i)], ov_hbm.at[sl(i)], sem_out.at[2 * i + 1]))
    for c in ins:
        c.start()
    for i in range(2 * _CHUNKS):
        ins[i].wait()
        outs[i].start()
    for c in outs:
        c.wait()


def kernel(k_new, v_new, k_cache, v_cache):
    del k_cache, v_cache
    shape = jax.ShapeDtypeStruct(k_new.shape, k_new.dtype)
    hbm = pl.BlockSpec(memory_space=pltpu.MemorySpace.HBM)
    out_k, out_v = pl.pallas_call(
        _copy_body,
        in_specs=[hbm, hbm],
        out_specs=[hbm, hbm],
        out_shape=[shape, shape],
        scratch_shapes=[
            pltpu.VMEM(shape.shape, shape.dtype),
            pltpu.VMEM(shape.shape, shape.dtype),
            pltpu.SemaphoreType.DMA((2 * _CHUNKS,)),
            pltpu.SemaphoreType.DMA((2 * _CHUNKS,)),
        ],
    )(k_new, v_new)
    return (out_k, out_v)


# speedup vs baseline: 1.0333x; 1.0333x over previous
import jax
import jax.numpy as jnp
from jax.experimental import pallas as pl
from jax.experimental.pallas import tpu as pltpu

_CHUNKS = 4
_ROWS = 32 // _CHUNKS


def _copy_body(k_hbm, v_hbm, ok_hbm, ov_hbm, kb, vb, sem_ik, sem_iv, sem_ok, sem_ov):
    def sl(i):
        return pl.ds(i * _ROWS, _ROWS)

    ins_k = [pltpu.make_async_copy(k_hbm.at[sl(i)], kb.at[sl(i)], sem_ik)
             for i in range(_CHUNKS)]
    ins_v = [pltpu.make_async_copy(v_hbm.at[sl(i)], vb.at[sl(i)], sem_iv)
             for i in range(_CHUNKS)]
    outs_k = [pltpu.make_async_copy(kb.at[sl(i)], ok_hbm.at[sl(i)], sem_ok)
              for i in range(_CHUNKS)]
    outs_v = [pltpu.make_async_copy(vb.at[sl(i)], ov_hbm.at[sl(i)], sem_ov)
              for i in range(_CHUNKS)]
    for i in range(_CHUNKS):
        ins_k[i].start()
        ins_v[i].start()
    for i in range(_CHUNKS):
        ins_k[i].wait()
        outs_k[i].start()
        ins_v[i].wait()
        outs_v[i].start()
    for i in range(_CHUNKS):
        outs_k[i].wait()
        outs_v[i].wait()


def kernel(k_new, v_new, k_cache, v_cache):
    del k_cache, v_cache
    shape = jax.ShapeDtypeStruct(k_new.shape, k_new.dtype)
    hbm = pl.BlockSpec(memory_space=pltpu.MemorySpace.HBM)
    out_k, out_v = pl.pallas_call(
        _copy_body,
        in_specs=[hbm, hbm],
        out_specs=[hbm, hbm],
        out_shape=[shape, shape],
        scratch_shapes=[
            pltpu.VMEM(shape.shape, shape.dtype),
            pltpu.VMEM(shape.shape, shape.dtype),
            pltpu.SemaphoreType.DMA,
            pltpu.SemaphoreType.DMA,
            pltpu.SemaphoreType.DMA,
            pltpu.SemaphoreType.DMA,
        ],
    )(k_new, v_new)
    return (out_k, out_v)
